# baseline (speedup 1.0000x reference)
import functools

import jax
import jax.numpy as jnp
from jax import lax
from jax.experimental import pallas as pl
from jax.experimental.pallas import tpu as pltpu
from jax.experimental.pallas import tpu_sc as plsc

GT = 512
TS = 512
TS_LOG2 = 9
CH = 64
NTILE48 = 48


def _gate_body(x_ref, wg_ref, bg_ref, tri_ref, idx_ref, gw_ref, rk_ref,
               offsp_ref, endt_ref, emap_ref, lb_ref, carry_ref, gsum_ref,
               *, nt, n_experts, n_tokens):
    t = pl.program_id(0)
    x = x_ref[...]
    logits = jnp.dot(x, wg_ref[...], preferred_element_type=jnp.float32)
    logits = logits + bg_ref[...]
    iota_e = jax.lax.broadcasted_iota(jnp.int32, logits.shape, 1)
    m1 = jnp.max(logits, axis=1, keepdims=True)
    idx1 = jnp.min(jnp.where(logits == m1, iota_e, n_experts), axis=1,
                   keepdims=True)
    l2 = jnp.where(iota_e == idx1, -jnp.inf, logits)
    m2 = jnp.max(l2, axis=1, keepdims=True)
    idx2 = jnp.min(jnp.where(l2 == m2, iota_e, n_experts), axis=1,
                   keepdims=True)
    e2 = jnp.exp(m2 - m1)
    g1 = 1.0 / (1.0 + e2)
    g2 = e2 * g1

    oh0 = (iota_e == idx1).astype(jnp.float32)
    oh1 = (iota_e == idx2).astype(jnp.float32)
    ohs = oh0 + oh1
    pre = jnp.dot(tri_ref[...], ohs, preferred_element_type=jnp.float32)
    carry = jnp.where(t == 0, jnp.zeros((1, n_experts), jnp.float32),
                      carry_ref[...])
    pc = pre + carry
    r0 = jnp.sum(pc * oh0, axis=1, keepdims=True)
    r1 = jnp.sum((pc + oh0) * oh1, axis=1, keepdims=True)
    new_carry = carry + jnp.sum(ohs, axis=0, keepdims=True)
    carry_ref[...] = new_carry

    idx_ref[...] = jnp.concatenate([idx1, idx2], axis=1)
    gw_ref[...] = jnp.concatenate([g1, g2], axis=1)
    rk_ref[...] = jnp.concatenate([r0, r1], axis=1).astype(jnp.int32)

    sg_sum = g1 * oh0 + g2 * oh1
    sg_sum = jnp.sum(sg_sum, axis=0, keepdims=True)
    prev = jnp.where(t == 0, jnp.zeros_like(sg_sum), gsum_ref[...])
    gsum_ref[...] = prev + sg_sum

    @pl.when(t == nt - 1)
    def _():
        cnt_i = new_carry.astype(jnp.int32)
        cpad = ((cnt_i + (TS - 1)) >> TS_LOG2) << TS_LOG2
        su = (jax.lax.broadcasted_iota(jnp.int32, (n_experts, n_experts), 0)
              < jax.lax.broadcasted_iota(jnp.int32, (n_experts, n_experts),
                                         1)).astype(jnp.float32)
        offs = jnp.dot(cpad.astype(jnp.float32), su,
                       preferred_element_type=jnp.float32).astype(jnp.int32)
        ends_real = offs + cnt_i
        ends_pad = offs + cpad

        offsp_ref[...] = jnp.broadcast_to(
            offs.reshape(n_experts, 1), (n_experts, 16))

        tids = jax.lax.broadcasted_iota(jnp.int32, (NTILE48, n_experts), 0)
        emap = jnp.sum((tids * TS >= ends_pad).astype(jnp.int32), axis=1,
                       keepdims=True)
        emap = jnp.minimum(emap, n_experts - 1)
        iota_te = jax.lax.broadcasted_iota(jnp.int32, (NTILE48, n_experts), 1)
        endt = jnp.sum(jnp.where(iota_te == emap, ends_real, 0), axis=1,
                       keepdims=True)
        endt_ref[...] = jnp.broadcast_to(endt, (NTILE48, 16))
        emap_ref[...] = jnp.broadcast_to(emap, (NTILE48, 16))

        d_i = gsum_ref[...] / n_tokens
        lb_ref[...] = jnp.sum(d_i * jnp.log(d_i + 1e-8), keepdims=True
                              ).reshape(1, 1)


def _gating(x, W_gate, b_gate):
    n, d = x.shape
    e_num = W_gate.shape[1]
    nt = n // GT
    tri = jnp.tril(jnp.ones((GT, GT), jnp.float32), -1)
    body = functools.partial(_gate_body, nt=nt, n_experts=e_num, n_tokens=n)
    const = lambda t: (0, 0)
    return pl.pallas_call(
        body,
        grid=(nt,),
        in_specs=[
            pl.BlockSpec((GT, d), lambda t: (t, 0)),
            pl.BlockSpec((d, e_num), const),
            pl.BlockSpec((1, e_num), const),
            pl.BlockSpec((GT, GT), const),
        ],
        out_specs=[
            pl.BlockSpec((GT, 2), lambda t: (t, 0)),
            pl.BlockSpec((GT, 2), lambda t: (t, 0)),
            pl.BlockSpec((GT, 2), lambda t: (t, 0)),
            pl.BlockSpec((e_num, 16), const),
            pl.BlockSpec((NTILE48, 16), const),
            pl.BlockSpec((NTILE48, 16), const),
            pl.BlockSpec((1, 1), const),
        ],
        out_shape=[
            jax.ShapeDtypeStruct((n, 2), jnp.int32),
            jax.ShapeDtypeStruct((n, 2), jnp.float32),
            jax.ShapeDtypeStruct((n, 2), jnp.int32),
            jax.ShapeDtypeStruct((e_num, 16), jnp.int32),
            jax.ShapeDtypeStruct((NTILE48, 16), jnp.int32),
            jax.ShapeDtypeStruct((NTILE48, 16), jnp.int32),
            jax.ShapeDtypeStruct((1, 1), jnp.float32),
        ],
        scratch_shapes=[
            pltpu.VMEM((1, e_num), jnp.float32),
            pltpu.VMEM((1, e_num), jnp.float32),
        ],
    )(x, W_gate, b_gate.reshape(1, e_num), tri)


def _make_scatter_pairs(npairs, npad, n_experts):
    mesh = plsc.VectorSubcoreMesh(core_axis_name="c", subcore_axis_name="s")
    per_w = npairs // 32
    nch = per_w // 128

    @functools.partial(
        pl.kernel, mesh=mesh,
        out_type=jax.ShapeDtypeStruct((npad, 128), jnp.int32),
        scratch_types=[
            pltpu.VMEM((per_w,), jnp.int32),
            pltpu.VMEM((per_w,), jnp.int32),
            pltpu.VMEM((n_experts, 16), jnp.int32),
            pltpu.VMEM((nch, 128), jnp.int32),
            pltpu.VMEM((128, 128), jnp.int32),
            pltpu.SemaphoreType.DMA,
        ],
    )
    def scatter_k(idx_hbm, rk_hbm, pack_hbm, offsp_hbm, out_hbm,
                  idxb, rkb, offsb, posb, pkb, sem):
        wid = lax.axis_index("s") * 2 + lax.axis_index("c")
        base = wid * per_w
        pltpu.sync_copy(idx_hbm.at[pl.ds(base, per_w)], idxb)
        pltpu.sync_copy(rk_hbm.at[pl.ds(base, per_w)], rkb)
        pltpu.sync_copy(offsp_hbm, offsb)
        off_rows = [offsb[e] for e in range(n_experts)]

        def chunk(ci, carry):
            for j in range(8):
                sl = pl.ds(ci * 128 + j * 16, 16)
                e = idxb[sl]
                pos = rkb[sl]
                for e2 in range(n_experts):
                    pos = pos + jnp.where(e == e2, off_rows[e2], 0)
                posb[ci, pl.ds(j * 16, 16)] = pos
            pltpu.sync_copy(pack_hbm.at[pl.ds(base + ci * 128, 128)], pkb)
            pltpu.async_copy(pkb, out_hbm.at[posb.at[ci]], sem).wait()
            return carry
        lax.fori_loop(0, nch, chunk, 0)

    return scatter_k


def _make_gather_rows(n, d, npad):
    mesh = plsc.VectorSubcoreMesh(core_axis_name="c", subcore_axis_name="s")
    per_w = npad // 32

    @functools.partial(
        pl.kernel, mesh=mesh,
        out_type=jax.ShapeDtypeStruct((npad, d), jnp.float32),
        scratch_types=[
            pltpu.VMEM((CH,), jnp.int32),
            pltpu.VMEM((CH,), jnp.int32),
            pltpu.VMEM((CH, d), jnp.float32),
            pltpu.SemaphoreType.DMA,
        ],
    )
    def gather_k(x_hbm, tok2_hbm, xg_hbm, tb, ib, rows, sem):
        wid = lax.axis_index("s") * 2 + lax.axis_index("c")
        base = wid * per_w

        def chunk(ci, carry):
            start = base + ci * CH
            pltpu.sync_copy(tok2_hbm.at[pl.ds(start, CH)], tb)
            for j in range(CH // 16):
                sl = pl.ds(j * 16, 16)
                tok = jnp.maximum(tb[sl], 0)
                ib[sl] = jnp.minimum(tok >> 1, n - 1)
            pltpu.async_copy(x_hbm.at[ib], rows, sem).wait()
            pltpu.sync_copy(rows, xg_hbm.at[pl.ds(start, CH)])
            return carry
        lax.fori_loop(0, per_w // CH, chunk, 0)

    return gather_k


def _ffn_body(emap_ref, xg_ref, gs_ref, w1_ref, b1_ref, w2_ref, b2_ref,
              y_ref):
    xb = xg_ref[...].astype(jnp.bfloat16)
    h = jnp.dot(xb, w1_ref[0], preferred_element_type=jnp.float32)
    h = jnp.maximum(h + b1_ref[0], 0.0)
    y = jnp.dot(h.astype(jnp.bfloat16), w2_ref[0],
                preferred_element_type=jnp.float32) + b2_ref[0]
    y_ref[...] = y * gs_ref[...]


def _ffn(e_map, xg, gsort2d, W1b, b1r, W2b, b2r):
    npad, d = xg.shape
    h_dim = W1b.shape[2]
    ntp = npad // TS
    grid_spec = pltpu.PrefetchScalarGridSpec(
        num_scalar_prefetch=1,
        grid=(ntp,),
        in_specs=[
            pl.BlockSpec((TS, d), lambda t, em: (t, 0)),
            pl.BlockSpec((TS, 1), lambda t, em: (t, 0)),
            pl.BlockSpec((1, d, h_dim), lambda t, em: (em[t], 0, 0)),
            pl.BlockSpec((1, 1, h_dim), lambda t, em: (em[t], 0, 0)),
            pl.BlockSpec((1, h_dim, d), lambda t, em: (em[t], 0, 0)),
            pl.BlockSpec((1, 1, d), lambda t, em: (em[t], 0, 0)),
        ],
        out_specs=pl.BlockSpec((TS, d), lambda t, em: (t, 0)),
    )
    return pl.pallas_call(
        _ffn_body,
        grid_spec=grid_spec,
        out_shape=jax.ShapeDtypeStruct((npad, d), jnp.float32),
    )(e_map, xg, gsort2d, W1b, b1r, W2b, b2r)


def _make_scatter_rows(d, npad, n2pad, npairs):
    mesh = plsc.VectorSubcoreMesh(core_axis_name="c", subcore_axis_name="s")
    per_w = npad // 32

    @functools.partial(
        pl.kernel, mesh=mesh,
        out_type=jax.ShapeDtypeStruct((n2pad, d), jnp.float32),
        scratch_types=[
            pltpu.VMEM((CH,), jnp.int32),
            pltpu.VMEM((CH,), jnp.int32),
            pltpu.VMEM((NTILE48, 16), jnp.int32),
            pltpu.VMEM((CH, d), jnp.float32),
            pltpu.SemaphoreType.DMA,
        ],
    )
    def scat_k(y_hbm, tok2_hbm, endt_hbm, y2_hbm, tb, ib, endsb, rows, sem):
        wid = lax.axis_index("s") * 2 + lax.axis_index("c")
        base = wid * per_w
        pltpu.sync_copy(endt_hbm, endsb)

        def chunk(ci, carry):
            start = base + ci * CH
            pltpu.sync_copy(tok2_hbm.at[pl.ds(start, CH)], tb)
            pltpu.sync_copy(y_hbm.at[pl.ds(start, CH)], rows)
            end_spl = endsb[start >> TS_LOG2]
            for j in range(CH // 16):
                sl = pl.ds(j * 16, 16)
                p_vec = lax.iota(jnp.int32, 16) + (start + j * 16)
                ib[sl] = jnp.where(p_vec < end_spl, tb[sl], npairs)
            pltpu.async_copy(rows, y2_hbm.at[ib], sem).wait()
            return carry
        lax.fori_loop(0, per_w // CH, chunk, 0)

    return scat_k


def _ln_body(x_ref, y2_ref, gamma_ref, beta_ref, out_ref):
    x = x_ref[...]
    d = x.shape[1]
    ys = y2_ref[...]
    y = x + ys[:, :d] + ys[:, d:]
    mu = jnp.mean(y, axis=1, keepdims=True)
    yc = y - mu
    var = jnp.mean(yc * yc, axis=1, keepdims=True)
    out_ref[...] = yc * jax.lax.rsqrt(var + 1e-5) * gamma_ref[...] \
        + beta_ref[...]


def _layernorm(x, y2flat, gamma, beta):
    n, d = x.shape
    nt = n // GT
    const = lambda t: (0, 0)
    return pl.pallas_call(
        _ln_body,
        grid=(nt,),
        in_specs=[
            pl.BlockSpec((GT, d), lambda t: (t, 0)),
            pl.BlockSpec((GT, 2 * d), lambda t: (t, 0)),
            pl.BlockSpec((1, d), const),
            pl.BlockSpec((1, d), const),
        ],
        out_specs=pl.BlockSpec((GT, d), lambda t: (t, 0)),
        out_shape=jax.ShapeDtypeStruct((n, d), jnp.float32),
    )(x, y2flat, gamma.reshape(1, d), beta.reshape(1, d))


def kernel(x, W_gate, b_gate, W1, b1, W2, b2, gamma, beta):
    n, d = x.shape
    e_num = W_gate.shape[1]
    h_dim = W1.shape[2]
    npairs = 2 * n
    npad = npairs + e_num * TS
    ntp = npad // TS
    n2pad = npairs + 8

    idx2, gw2, rk2, offsplat, endtile, emap48, lb = _gating(x, W_gate, b_gate)

    pack8 = jnp.concatenate(
        [jnp.arange(npairs, dtype=jnp.int32).reshape(npairs, 1),
         jax.lax.bitcast_convert_type(gw2.reshape(npairs, 1), jnp.int32),
         jnp.zeros((npairs, 126), jnp.int32)], axis=1)

    packed = _make_scatter_pairs(npairs, npad, e_num)(
        idx2.reshape(npairs), rk2.reshape(npairs), pack8, offsplat)
    tok2f = packed[:, 0]
    gsort2d = jax.lax.bitcast_convert_type(packed[:, 1:2], jnp.float32)

    xg = _make_gather_rows(n, d, npad)(x, tok2f)

    e_map = emap48[:ntp, 0]

    ys = _ffn(e_map, xg, gsort2d,
              W1.astype(jnp.bfloat16), b1.reshape(e_num, 1, h_dim),
              W2.astype(jnp.bfloat16), b2.reshape(e_num, 1, d))

    y2 = _make_scatter_rows(d, npad, n2pad, npairs)(ys, tok2f, endtile)

    out = _layernorm(x, y2[:npairs].reshape(n, 2 * d), gamma, beta)
    return out, lb[0, 0]


# speedup vs baseline: 1.0110x; 1.0110x over previous
import functools

import jax
import jax.numpy as jnp
from jax import lax
from jax.experimental import pallas as pl
from jax.experimental.pallas import tpu as pltpu
from jax.experimental.pallas import tpu_sc as plsc

GT = 512
TS = 512
TS_LOG2 = 9
CH = 128
NTILE48 = 48


def _gate_body(x_ref, wg_ref, bg_ref, tri_ref, idx_ref, gw_ref, rk_ref,
               offsp_ref, endt_ref, emap_ref, lb_ref, carry_ref, gsum_ref,
               *, nt, n_experts, n_tokens):
    t = pl.program_id(0)
    x = x_ref[...]
    logits = jnp.dot(x, wg_ref[...], preferred_element_type=jnp.float32)
    logits = logits + bg_ref[...]
    iota_e = jax.lax.broadcasted_iota(jnp.int32, logits.shape, 1)
    m1 = jnp.max(logits, axis=1, keepdims=True)
    idx1 = jnp.min(jnp.where(logits == m1, iota_e, n_experts), axis=1,
                   keepdims=True)
    l2 = jnp.where(iota_e == idx1, -jnp.inf, logits)
    m2 = jnp.max(l2, axis=1, keepdims=True)
    idx2 = jnp.min(jnp.where(l2 == m2, iota_e, n_experts), axis=1,
                   keepdims=True)
    e2 = jnp.exp(m2 - m1)
    g1 = 1.0 / (1.0 + e2)
    g2 = e2 * g1

    oh0 = (iota_e == idx1).astype(jnp.float32)
    oh1 = (iota_e == idx2).astype(jnp.float32)
    ohs = oh0 + oh1
    pre = jnp.dot(tri_ref[...], ohs, preferred_element_type=jnp.float32)
    carry = jnp.where(t == 0, jnp.zeros((1, n_experts), jnp.float32),
                      carry_ref[...])
    pc = pre + carry
    r0 = jnp.sum(pc * oh0, axis=1, keepdims=True)
    r1 = jnp.sum((pc + oh0) * oh1, axis=1, keepdims=True)
    new_carry = carry + jnp.sum(ohs, axis=0, keepdims=True)
    carry_ref[...] = new_carry

    idx_ref[...] = jnp.concatenate([idx1, idx2], axis=1)
    gw_ref[...] = jnp.concatenate([g1, g2], axis=1)
    rk_ref[...] = jnp.concatenate([r0, r1], axis=1).astype(jnp.int32)

    sg_sum = g1 * oh0 + g2 * oh1
    sg_sum = jnp.sum(sg_sum, axis=0, keepdims=True)
    prev = jnp.where(t == 0, jnp.zeros_like(sg_sum), gsum_ref[...])
    gsum_ref[...] = prev + sg_sum

    @pl.when(t == nt - 1)
    def _():
        cnt_i = new_carry.astype(jnp.int32)
        cpad = ((cnt_i + (TS - 1)) >> TS_LOG2) << TS_LOG2
        su = (jax.lax.broadcasted_iota(jnp.int32, (n_experts, n_experts), 0)
              < jax.lax.broadcasted_iota(jnp.int32, (n_experts, n_experts),
                                         1)).astype(jnp.float32)
        offs = jnp.dot(cpad.astype(jnp.float32), su,
                       preferred_element_type=jnp.float32).astype(jnp.int32)
        ends_real = offs + cnt_i
        ends_pad = offs + cpad

        offsp_ref[...] = jnp.broadcast_to(
            offs.reshape(n_experts, 1), (n_experts, 16))

        tids = jax.lax.broadcasted_iota(jnp.int32, (NTILE48, n_experts), 0)
        emap = jnp.sum((tids * TS >= ends_pad).astype(jnp.int32), axis=1,
                       keepdims=True)
        emap = jnp.minimum(emap, n_experts - 1)
        iota_te = jax.lax.broadcasted_iota(jnp.int32, (NTILE48, n_experts), 1)
        endt = jnp.sum(jnp.where(iota_te == emap, ends_real, 0), axis=1,
                       keepdims=True)
        endt_ref[...] = jnp.broadcast_to(endt, (NTILE48, 16))
        emap_ref[...] = jnp.broadcast_to(emap, (NTILE48, 16))

        d_i = gsum_ref[...] / n_tokens
        lb_ref[...] = jnp.sum(d_i * jnp.log(d_i + 1e-8), keepdims=True
                              ).reshape(1, 1)


def _gating(x, W_gate, b_gate):
    n, d = x.shape
    e_num = W_gate.shape[1]
    nt = n // GT
    tri = jnp.tril(jnp.ones((GT, GT), jnp.float32), -1)
    body = functools.partial(_gate_body, nt=nt, n_experts=e_num, n_tokens=n)
    const = lambda t: (0, 0)
    return pl.pallas_call(
        body,
        grid=(nt,),
        in_specs=[
            pl.BlockSpec((GT, d), lambda t: (t, 0)),
            pl.BlockSpec((d, e_num), const),
            pl.BlockSpec((1, e_num), const),
            pl.BlockSpec((GT, GT), const),
        ],
        out_specs=[
            pl.BlockSpec((GT, 2), lambda t: (t, 0)),
            pl.BlockSpec((GT, 2), lambda t: (t, 0)),
            pl.BlockSpec((GT, 2), lambda t: (t, 0)),
            pl.BlockSpec((e_num, 16), const),
            pl.BlockSpec((NTILE48, 16), const),
            pl.BlockSpec((NTILE48, 16), const),
            pl.BlockSpec((1, 1), const),
        ],
        out_shape=[
            jax.ShapeDtypeStruct((n, 2), jnp.int32),
            jax.ShapeDtypeStruct((n, 2), jnp.float32),
            jax.ShapeDtypeStruct((n, 2), jnp.int32),
            jax.ShapeDtypeStruct((e_num, 16), jnp.int32),
            jax.ShapeDtypeStruct((NTILE48, 16), jnp.int32),
            jax.ShapeDtypeStruct((NTILE48, 16), jnp.int32),
            jax.ShapeDtypeStruct((1, 1), jnp.float32),
        ],
        scratch_shapes=[
            pltpu.VMEM((1, e_num), jnp.float32),
            pltpu.VMEM((1, e_num), jnp.float32),
        ],
    )(x, W_gate, b_gate.reshape(1, e_num), tri)


def _make_scatter_pairs(npairs, npad, n_experts):
    mesh = plsc.VectorSubcoreMesh(core_axis_name="c", subcore_axis_name="s")
    per_w = npairs // 32
    nch = per_w // 128

    @functools.partial(
        pl.kernel, mesh=mesh,
        out_type=jax.ShapeDtypeStruct((npad, 128), jnp.int32),
        scratch_types=[
            pltpu.VMEM((per_w,), jnp.int32),
            pltpu.VMEM((per_w,), jnp.int32),
            pltpu.VMEM((n_experts, 16), jnp.int32),
            pltpu.VMEM((nch, 128), jnp.int32),
            pltpu.VMEM((128, 128), jnp.int32),
            pltpu.SemaphoreType.DMA,
        ],
    )
    def scatter_k(idx_hbm, rk_hbm, pack_hbm, offsp_hbm, out_hbm,
                  idxb, rkb, offsb, posb, pkb, sem):
        wid = lax.axis_index("s") * 2 + lax.axis_index("c")
        base = wid * per_w
        pltpu.sync_copy(idx_hbm.at[pl.ds(base, per_w)], idxb)
        pltpu.sync_copy(rk_hbm.at[pl.ds(base, per_w)], rkb)
        pltpu.sync_copy(offsp_hbm, offsb)
        off_rows = [offsb[e] for e in range(n_experts)]

        def chunk(ci, carry):
            for j in range(8):
                sl = pl.ds(ci * 128 + j * 16, 16)
                e = idxb[sl]
                pos = rkb[sl]
                for e2 in range(n_experts):
                    pos = pos + jnp.where(e == e2, off_rows[e2], 0)
                posb[ci, pl.ds(j * 16, 16)] = pos
            pltpu.sync_copy(pack_hbm.at[pl.ds(base + ci * 128, 128)], pkb)
            pltpu.async_copy(pkb, out_hbm.at[posb.at[ci]], sem).wait()
            return carry
        lax.fori_loop(0, nch, chunk, 0)

    return scatter_k


def _make_gather_rows(n, d, npad):
    mesh = plsc.VectorSubcoreMesh(core_axis_name="c", subcore_axis_name="s")
    per_w = npad // 32

    @functools.partial(
        pl.kernel, mesh=mesh,
        out_type=jax.ShapeDtypeStruct((npad, d), jnp.float32),
        scratch_types=[
            pltpu.VMEM((CH,), jnp.int32),
            pltpu.VMEM((CH,), jnp.int32),
            pltpu.VMEM((CH, d), jnp.float32),
            pltpu.SemaphoreType.DMA,
        ],
    )
    def gather_k(x_hbm, tok2_hbm, xg_hbm, tb, ib, rows, sem):
        wid = lax.axis_index("s") * 2 + lax.axis_index("c")
        base = wid * per_w

        def chunk(ci, carry):
            start = base + ci * CH
            pltpu.sync_copy(tok2_hbm.at[pl.ds(start, CH)], tb)
            for j in range(CH // 16):
                sl = pl.ds(j * 16, 16)
                tok = jnp.maximum(tb[sl], 0)
                ib[sl] = jnp.minimum(tok >> 1, n - 1)
            pltpu.async_copy(x_hbm.at[ib], rows, sem).wait()
            pltpu.sync_copy(rows, xg_hbm.at[pl.ds(start, CH)])
            return carry
        lax.fori_loop(0, per_w // CH, chunk, 0)

    return gather_k


def _ffn_body(emap_ref, xg_ref, gs_ref, w1_ref, b1_ref, w2_ref, b2_ref,
              y_ref):
    xb = xg_ref[...].astype(jnp.bfloat16)
    h = jnp.dot(xb, w1_ref[0], preferred_element_type=jnp.float32)
    h = jnp.maximum(h + b1_ref[0], 0.0)
    y = jnp.dot(h.astype(jnp.bfloat16), w2_ref[0],
                preferred_element_type=jnp.float32) + b2_ref[0]
    y_ref[...] = y * gs_ref[...]


def _ffn(e_map, xg, gsort2d, W1b, b1r, W2b, b2r):
    npad, d = xg.shape
    h_dim = W1b.shape[2]
    ntp = npad // TS
    grid_spec = pltpu.PrefetchScalarGridSpec(
        num_scalar_prefetch=1,
        grid=(ntp,),
        in_specs=[
            pl.BlockSpec((TS, d), lambda t, em: (t, 0)),
            pl.BlockSpec((TS, 1), lambda t, em: (t, 0)),
            pl.BlockSpec((1, d, h_dim), lambda t, em: (em[t], 0, 0)),
            pl.BlockSpec((1, 1, h_dim), lambda t, em: (em[t], 0, 0)),
            pl.BlockSpec((1, h_dim, d), lambda t, em: (em[t], 0, 0)),
            pl.BlockSpec((1, 1, d), lambda t, em: (em[t], 0, 0)),
        ],
        out_specs=pl.BlockSpec((TS, d), lambda t, em: (t, 0)),
    )
    return pl.pallas_call(
        _ffn_body,
        grid_spec=grid_spec,
        out_shape=jax.ShapeDtypeStruct((npad, d), jnp.float32),
    )(e_map, xg, gsort2d, W1b, b1r, W2b, b2r)


def _make_scatter_rows(d, npad, n2pad, npairs):
    mesh = plsc.VectorSubcoreMesh(core_axis_name="c", subcore_axis_name="s")
    per_w = npad // 32

    @functools.partial(
        pl.kernel, mesh=mesh,
        out_type=jax.ShapeDtypeStruct((n2pad, d), jnp.float32),
        scratch_types=[
            pltpu.VMEM((CH,), jnp.int32),
            pltpu.VMEM((CH,), jnp.int32),
            pltpu.VMEM((NTILE48, 16), jnp.int32),
            pltpu.VMEM((CH, d), jnp.float32),
            pltpu.SemaphoreType.DMA,
        ],
    )
    def scat_k(y_hbm, tok2_hbm, endt_hbm, y2_hbm, tb, ib, endsb, rows, sem):
        wid = lax.axis_index("s") * 2 + lax.axis_index("c")
        base = wid * per_w
        pltpu.sync_copy(endt_hbm, endsb)

        def chunk(ci, carry):
            start = base + ci * CH
            pltpu.sync_copy(tok2_hbm.at[pl.ds(start, CH)], tb)
            pltpu.sync_copy(y_hbm.at[pl.ds(start, CH)], rows)
            end_spl = endsb[start >> TS_LOG2]
            for j in range(CH // 16):
                sl = pl.ds(j * 16, 16)
                p_vec = lax.iota(jnp.int32, 16) + (start + j * 16)
                ib[sl] = jnp.where(p_vec < end_spl, tb[sl], npairs)
            pltpu.async_copy(rows, y2_hbm.at[ib], sem).wait()
            return carry
        lax.fori_loop(0, per_w // CH, chunk, 0)

    return scat_k


def _ln_body(x_ref, y2_ref, gamma_ref, beta_ref, out_ref):
    x = x_ref[...]
    d = x.shape[1]
    ys = y2_ref[...]
    y = x + ys[:, :d] + ys[:, d:]
    mu = jnp.mean(y, axis=1, keepdims=True)
    yc = y - mu
    var = jnp.mean(yc * yc, axis=1, keepdims=True)
    out_ref[...] = yc * jax.lax.rsqrt(var + 1e-5) * gamma_ref[...] \
        + beta_ref[...]


def _layernorm(x, y2flat, gamma, beta):
    n, d = x.shape
    nt = n // GT
    const = lambda t: (0, 0)
    return pl.pallas_call(
        _ln_body,
        grid=(nt,),
        in_specs=[
            pl.BlockSpec((GT, d), lambda t: (t, 0)),
            pl.BlockSpec((GT, 2 * d), lambda t: (t, 0)),
            pl.BlockSpec((1, d), const),
            pl.BlockSpec((1, d), const),
        ],
        out_specs=pl.BlockSpec((GT, d), lambda t: (t, 0)),
        out_shape=jax.ShapeDtypeStruct((n, d), jnp.float32),
    )(x, y2flat, gamma.reshape(1, d), beta.reshape(1, d))


def kernel(x, W_gate, b_gate, W1, b1, W2, b2, gamma, beta):
    n, d = x.shape
    e_num = W_gate.shape[1]
    h_dim = W1.shape[2]
    npairs = 2 * n
    npad = npairs + e_num * TS
    ntp = npad // TS
    n2pad = npairs + 8

    idx2, gw2, rk2, offsplat, endtile, emap48, lb = _gating(x, W_gate, b_gate)

    pack8 = jnp.concatenate(
        [jnp.arange(npairs, dtype=jnp.int32).reshape(npairs, 1),
         jax.lax.bitcast_convert_type(gw2.reshape(npairs, 1), jnp.int32),
         jnp.zeros((npairs, 126), jnp.int32)], axis=1)

    packed = _make_scatter_pairs(npairs, npad, e_num)(
        idx2.reshape(npairs), rk2.reshape(npairs), pack8, offsplat)
    tok2f = packed[:, 0]
    gsort2d = jax.lax.bitcast_convert_type(packed[:, 1:2], jnp.float32)

    xg = _make_gather_rows(n, d, npad)(x, tok2f)

    e_map = emap48[:ntp, 0]

    ys = _ffn(e_map, xg, gsort2d,
              W1.astype(jnp.bfloat16), b1.reshape(e_num, 1, h_dim),
              W2.astype(jnp.bfloat16), b2.reshape(e_num, 1, d))

    y2 = _make_scatter_rows(d, npad, n2pad, npairs)(ys, tok2f, endtile)

    out = _layernorm(x, y2[:npairs].reshape(n, 2 * d), gamma, beta)
    return out, lb[0, 0]


# speedup vs baseline: 3.4695x; 3.4317x over previous
import functools

import jax
import jax.numpy as jnp
from jax.experimental import pallas as pl
from jax.experimental.pallas import tpu as pltpu

TILE_N = 1024


def _moe_body(x_ref, wg_ref, bg_ref, w1c_ref, b1c_ref, w2c_ref, b2_ref,
              gamma_ref, beta_ref, out_ref, lb_ref, gsum_ref,
              *, nt, n_experts, n_tokens):
    t = pl.program_id(0)
    x = x_ref[...]

    logits = jnp.dot(x, wg_ref[...], preferred_element_type=jnp.float32)
    logits = logits + bg_ref[...]
    iota_e = jax.lax.broadcasted_iota(jnp.int32, logits.shape, 1)
    m1 = jnp.max(logits, axis=1, keepdims=True)
    idx1 = jnp.min(jnp.where(logits == m1, iota_e, n_experts), axis=1,
                   keepdims=True)
    l2 = jnp.where(iota_e == idx1, -jnp.inf, logits)
    m2 = jnp.max(l2, axis=1, keepdims=True)
    idx2 = jnp.min(jnp.where(l2 == m2, iota_e, n_experts), axis=1,
                   keepdims=True)
    e2 = jnp.exp(m2 - m1)
    g1 = 1.0 / (1.0 + e2)
    g2 = e2 * g1
    gate_s = g1 * (iota_e == idx1) + g2 * (iota_e == idx2)

    sg_sum = jnp.sum(gate_s, axis=0, keepdims=True)
    prev = jnp.where(t == 0, jnp.zeros_like(sg_sum), gsum_ref[...])
    gsum_ref[...] = prev + sg_sum

    xb = x.astype(jnp.bfloat16)
    h = jnp.dot(xb, w1c_ref[...], preferred_element_type=jnp.float32)
    hb = h.astype(jnp.bfloat16)
    hb = jnp.maximum(hb + b1c_ref[...], jnp.bfloat16(0.0))
    h_dim = hb.shape[1] // n_experts
    chunks = []
    for e in range(n_experts):
        ge = (g1 * (idx1 == e) + g2 * (idx2 == e)).astype(jnp.bfloat16)
        chunks.append(hb[:, e * h_dim:(e + 1) * h_dim] * ge)
    gh = jnp.concatenate(chunks, axis=1)
    moe = jnp.dot(gh, w2c_ref[...], preferred_element_type=jnp.float32)
    moe = moe + jnp.dot(gate_s, b2_ref[...],
                        preferred_element_type=jnp.float32)

    y = moe + x
    mu = jnp.mean(y, axis=1, keepdims=True)
    yc = y - mu
    var = jnp.mean(yc * yc, axis=1, keepdims=True)
    out_ref[...] = yc * jax.lax.rsqrt(var + 1e-5) * gamma_ref[...] \
        + beta_ref[...]

    @pl.when(t == nt - 1)
    def _():
        d_i = gsum_ref[...] / n_tokens
        lb_ref[...] = jnp.sum(d_i * jnp.log(d_i + 1e-8), keepdims=True
                              ).reshape(1, 1)


def kernel(x, W_gate, b_gate, W1, b1, W2, b2, gamma, beta):
    n, d = x.shape
    e_num = W_gate.shape[1]
    h_dim = W1.shape[2]
    eh = e_num * h_dim
    nt = n // TILE_N

    w1c = jnp.transpose(W1, (1, 0, 2)).reshape(d, eh).astype(jnp.bfloat16)
    b1c = b1.reshape(1, eh).astype(jnp.bfloat16)
    w2c = W2.reshape(eh, d).astype(jnp.bfloat16)

    body = functools.partial(_moe_body, nt=nt, n_experts=e_num, n_tokens=n)
    const = lambda t: (0, 0)
    out, lb = pl.pallas_call(
        body,
        grid=(nt,),
        in_specs=[
            pl.BlockSpec((TILE_N, d), lambda t: (t, 0)),
            pl.BlockSpec((d, e_num), const),
            pl.BlockSpec((1, e_num), const),
            pl.BlockSpec((d, eh), const),
            pl.BlockSpec((1, eh), const),
            pl.BlockSpec((eh, d), const),
            pl.BlockSpec((e_num, d), const),
            pl.BlockSpec((1, d), const),
            pl.BlockSpec((1, d), const),
        ],
        out_specs=[
            pl.BlockSpec((TILE_N, d), lambda t: (t, 0)),
            pl.BlockSpec((1, 1), const),
        ],
        out_shape=[
            jax.ShapeDtypeStruct((n, d), jnp.float32),
            jax.ShapeDtypeStruct((1, 1), jnp.float32),
        ],
        scratch_shapes=[
            pltpu.VMEM((1, e_num), jnp.float32),
        ],
    )(x, W_gate, b_gate.reshape(1, e_num), w1c, b1c, w2c, b2,
      gamma.reshape(1, d), beta.reshape(1, d))
    return out, lb[0, 0]


# speedup vs baseline: 3.4863x; 1.0049x over previous
import functools

import jax
import jax.numpy as jnp
from jax.experimental import pallas as pl
from jax.experimental.pallas import tpu as pltpu

TILE_N = 1024


def _moe_half(x, wg, bg, w1c, b1c, w2c, b2, gamma, beta, n_experts):
    logits = jnp.dot(x, wg, preferred_element_type=jnp.float32)
    logits = logits + bg
    iota_e = jax.lax.broadcasted_iota(jnp.int32, logits.shape, 1)
    m1 = jnp.max(logits, axis=1, keepdims=True)
    idx1 = jnp.min(jnp.where(logits == m1, iota_e, n_experts), axis=1,
                   keepdims=True)
    l2 = jnp.where(iota_e == idx1, -jnp.inf, logits)
    m2 = jnp.max(l2, axis=1, keepdims=True)
    idx2 = jnp.min(jnp.where(l2 == m2, iota_e, n_experts), axis=1,
                   keepdims=True)
    e2 = jnp.exp(m2 - m1)
    g1 = 1.0 / (1.0 + e2)
    g2 = e2 * g1
    gate_s = g1 * (iota_e == idx1) + g2 * (iota_e == idx2)
    sg_sum = jnp.sum(gate_s, axis=0, keepdims=True)

    xb = x.astype(jnp.bfloat16)
    h = jnp.dot(xb, w1c, preferred_element_type=jnp.float32)
    hb = h.astype(jnp.bfloat16)
    hb = jnp.maximum(hb + b1c, jnp.bfloat16(0.0))
    h_dim = hb.shape[1] // n_experts
    chunks = []
    for e in range(n_experts):
        ge = (g1 * (idx1 == e) + g2 * (idx2 == e)).astype(jnp.bfloat16)
        chunks.append(hb[:, e * h_dim:(e + 1) * h_dim] * ge)
    gh = jnp.concatenate(chunks, axis=1)
    moe = jnp.dot(gh, w2c, preferred_element_type=jnp.float32)
    moe = moe + jnp.dot(gate_s, b2, preferred_element_type=jnp.float32)

    y = moe + x
    mu = jnp.mean(y, axis=1, keepdims=True)
    yc = y - mu
    var = jnp.mean(yc * yc, axis=1, keepdims=True)
    out = yc * jax.lax.rsqrt(var + 1e-5) * gamma + beta
    return out, sg_sum


def _moe_body(x_ref, wg_ref, bg_ref, w1c_ref, b1c_ref, w2c_ref, b2_ref,
              gamma_ref, beta_ref, out_ref, lb_ref, gsum_ref,
              *, nt, n_experts, n_tokens, n_halves):
    t = pl.program_id(0)
    half = x_ref.shape[0] // n_halves
    args = (wg_ref[...], bg_ref[...], w1c_ref[...], b1c_ref[...],
            w2c_ref[...], b2_ref[...], gamma_ref[...], beta_ref[...])
    sg_total = None
    for p in range(n_halves):
        sl = slice(p * half, (p + 1) * half)
        out, sg = _moe_half(x_ref[sl, :], *args, n_experts)
        out_ref[sl, :] = out
        sg_total = sg if sg_total is None else sg_total + sg

    prev = jnp.where(t == 0, jnp.zeros_like(sg_total), gsum_ref[...])
    gsum_ref[...] = prev + sg_total

    @pl.when(t == nt - 1)
    def _():
        d_i = gsum_ref[...] / n_tokens
        lb_ref[...] = jnp.sum(d_i * jnp.log(d_i + 1e-8), keepdims=True
                              ).reshape(1, 1)


def kernel(x, W_gate, b_gate, W1, b1, W2, b2, gamma, beta):
    n, d = x.shape
    e_num = W_gate.shape[1]
    h_dim = W1.shape[2]
    eh = e_num * h_dim
    nt = n // TILE_N

    w1c = jnp.transpose(W1, (1, 0, 2)).reshape(d, eh).astype(jnp.bfloat16)
    b1c = b1.reshape(1, eh).astype(jnp.bfloat16)
    w2c = W2.reshape(eh, d).astype(jnp.bfloat16)

    body = functools.partial(_moe_body, nt=nt, n_experts=e_num, n_tokens=n, n_halves=2)
    const = lambda t: (0, 0)
    out, lb = pl.pallas_call(
        body,
        grid=(nt,),
        in_specs=[
            pl.BlockSpec((TILE_N, d), lambda t: (t, 0)),
            pl.BlockSpec((d, e_num), const),
            pl.BlockSpec((1, e_num), const),
            pl.BlockSpec((d, eh), const),
            pl.BlockSpec((1, eh), const),
            pl.BlockSpec((eh, d), const),
            pl.BlockSpec((e_num, d), const),
            pl.BlockSpec((1, d), const),
            pl.BlockSpec((1, d), const),
        ],
        out_specs=[
            pl.BlockSpec((TILE_N, d), lambda t: (t, 0)),
            pl.BlockSpec((1, 1), const),
        ],
        out_shape=[
            jax.ShapeDtypeStruct((n, d), jnp.float32),
            jax.ShapeDtypeStruct((1, 1), jnp.float32),
        ],
        scratch_shapes=[
            pltpu.VMEM((1, e_num), jnp.float32),
        ],
    )(x, W_gate, b_gate.reshape(1, e_num), w1c, b1c, w2c, b2,
      gamma.reshape(1, d), beta.reshape(1, d))
    return out, lb[0, 0]
